# baseline (speedup 1.0000x reference)
import functools

import jax
import jax.numpy as jnp
from jax import lax
from jax.experimental import pallas as pl
from jax.experimental.pallas import tpu as pltpu
from jax.experimental.pallas import tpu_sc as plsc

_N = 16384
_C = 1000

_SC_TILES_PER_TEC = 2
_NW = 32
_SC_ROWS = _SC_TILES_PER_TEC * 128 * _NW
_TC_ROWS = _N - _SC_ROWS

_BN = 2048
_NB = _TC_ROWS // _BN
_OFF = _SC_ROWS // _BN
_TR = _BN // 128
_TOFF = _SC_ROWS // 128

_CC = 200
_NCC = _C // _CC


def _tc_kernel(x_ref, t_ref, out_ref, acc_ref):
    i = pl.program_id(0)
    x = x_ref[...]
    parts = []
    for j in range(_TR):
        xc = x[:, j * 128:(j + 1) * 128]
        row = jax.lax.broadcasted_iota(jnp.int32, xc.shape, 0)
        mx = jnp.max(xc, axis=0, keepdims=True)
        parts.append(jnp.min(jnp.where(xc == mx, row, _C), axis=0, keepdims=True))
    pred = jnp.concatenate(parts, axis=0)
    tgt = t_ref[...]
    m = (pred != tgt).astype(jnp.float32)
    pf = pred.astype(jnp.float32)
    pm = pf * m
    bn = jnp.sum(m)
    bs1 = jnp.sum(pm)
    bs2 = jnp.sum(pf * pm)

    @pl.when(i == 0)
    def _():
        acc_ref[0] = bn
        acc_ref[1] = bs1
        acc_ref[2] = bs2

    @pl.when(i != 0)
    def _():
        acc_ref[0] += bn
        acc_ref[1] += bs1
        acc_ref[2] += bs2

    @pl.when(i == _NB - 1)
    def _():
        out_ref[0, 0] = acc_ref[0]
        out_ref[0, 1] = acc_ref[1]
        out_ref[0, 2] = acc_ref[2]


def _tc_partials(xt, t128):
    return pl.pallas_call(
        _tc_kernel,
        grid=(_NB,),
        in_specs=[
            pl.BlockSpec((_C, _BN), lambda i: (0, i + _OFF)),
            pl.BlockSpec((_TR, 128), lambda i: (i + _TOFF // _TR, 0)),
        ],
        out_specs=pl.BlockSpec(
            (1, 3), lambda i: (0, 0), memory_space=pltpu.SMEM
        ),
        out_shape=jax.ShapeDtypeStruct((1, 3), jnp.float32),
        scratch_shapes=[pltpu.SMEM((3,), jnp.float32)],
        compiler_params=pltpu.CompilerParams(
            dimension_semantics=("arbitrary",),
        ),
    )(xt, t128)


def _sc_partials(xt, targets):
    mesh = plsc.VectorSubcoreMesh(core_axis_name="c", subcore_axis_name="s")

    @functools.partial(
        pl.kernel,
        mesh=mesh,
        out_type=jax.ShapeDtypeStruct((_NW * 48,), jnp.float32),
        scratch_types=[
            pltpu.VMEM((_CC, 128), jnp.float32),
            pltpu.VMEM((_CC, 128), jnp.float32),
            pltpu.VMEM((128,), jnp.int32),
            pltpu.VMEM((48,), jnp.float32),
            pltpu.SemaphoreType.DMA,
            pltpu.SemaphoreType.DMA,
            pltpu.SemaphoreType.DMA,
        ],
    )
    def k(xt_hbm, t_hbm, out_hbm, buf_a, buf_b, tvec, part, sem_a, sem_b, sem_t):
        cid = lax.axis_index("c")
        sid = lax.axis_index("s")
        wid = sid * 2 + cid
        bufs = (buf_a, buf_b)
        sems = (sem_a, sem_b)

        n_acc = jnp.zeros((16,), jnp.float32)
        s1_acc = jnp.zeros((16,), jnp.float32)
        s2_acc = jnp.zeros((16,), jnp.float32)

        for k_t in range(_SC_TILES_PER_TEC):
            tile = wid * _SC_TILES_PER_TEC + k_t
            base = tile * 128
            t_h = pltpu.async_copy(
                t_hbm.at[pl.ds(base, 128)], tvec, sem_t
            )
            handles = [None, None]
            handles[0] = pltpu.async_copy(
                xt_hbm.at[pl.ds(0, _CC), pl.ds(base, 128)], buf_a, sem_a
            )
            bests = [
                (jnp.full((16,), -jnp.inf, jnp.float32),
                 jnp.zeros((16,), jnp.float32))
                for _ in range(8)
            ]
            for cc in range(_NCC):
                cur = cc % 2
                if cc + 1 < _NCC:
                    nxt = (cc + 1) % 2
                    handles[nxt] = pltpu.async_copy(
                        xt_hbm.at[pl.ds((cc + 1) * _CC, _CC), pl.ds(base, 128)],
                        bufs[nxt], sems[nxt],
                    )
                handles[cur].wait()
                buf = bufs[cur]
                for g in range(8):
                    best, bidx = bests[g]

                    def body(j, carry, _g=g, _buf=buf, _cc=cc):
                        b, bi = carry
                        for u in range(8):
                            c_loc = j * 8 + u
                            v = _buf[c_loc, pl.ds(_g * 16, 16)]
                            cf = (_cc * _CC + c_loc).astype(jnp.float32)
                            cv = jnp.full((16,), 0.0, jnp.float32) + cf
                            upd = v > b
                            b = jnp.where(upd, v, b)
                            bi = jnp.where(upd, cv, bi)
                        return b, bi

                    best, bidx = lax.fori_loop(
                        0, _CC // 8, body, (best, bidx)
                    )
                    bests[g] = (best, bidx)
            t_h.wait()
            for g in range(8):
                best, bidx = bests[g]
                tg = tvec[pl.ds(g * 16, 16)].astype(jnp.float32)
                m = jnp.where(bidx != tg, 1.0, 0.0)
                pm = bidx * m
                n_acc = n_acc + m
                s1_acc = s1_acc + pm
                s2_acc = s2_acc + bidx * pm

        part[pl.ds(0, 16)] = n_acc
        part[pl.ds(16, 16)] = s1_acc
        part[pl.ds(32, 16)] = s2_acc
        pltpu.sync_copy(part, out_hbm.at[pl.ds(wid * 48, 48)])

    return k(xt, targets)


def _combine_kernel(tc_ref, sc_ref, out_ref):
    sc = sc_ref[...]
    n = tc_ref[0, 0]
    s1 = tc_ref[0, 1]
    s2 = tc_ref[0, 2]
    nv = jnp.zeros((16,), jnp.float32)
    s1v = jnp.zeros((16,), jnp.float32)
    s2v = jnp.zeros((16,), jnp.float32)
    for w in range(_NW):
        nv = nv + lax.slice(sc, (w * 48,), (w * 48 + 16,))
        s1v = s1v + lax.slice(sc, (w * 48 + 16,), (w * 48 + 32,))
        s2v = s2v + lax.slice(sc, (w * 48 + 32,), (w * 48 + 48,))
    n = n + jnp.sum(nv)
    s1 = s1 + jnp.sum(s1v)
    s2 = s2 + jnp.sum(s2v)
    mean = s1 / n
    var = (s2 - s1 * mean) / (n - 1.0)
    out_ref[0, 0] = 1.0 - jnp.sqrt(var)


def _combine(tc_part, sc_part):
    return pl.pallas_call(
        _combine_kernel,
        in_specs=[
            pl.BlockSpec(memory_space=pltpu.SMEM),
            pl.BlockSpec(memory_space=pltpu.VMEM),
        ],
        out_specs=pl.BlockSpec(memory_space=pltpu.SMEM),
        out_shape=jax.ShapeDtypeStruct((1, 1), jnp.float32),
    )(tc_part, sc_part)


def kernel(inputs, targets):
    xt = inputs.T
    t128 = targets.reshape(128, 128)
    sc_part = _sc_partials(xt, targets)
    tc_part = _tc_partials(xt, t128)
    out = _combine(tc_part, sc_part)
    return out.reshape(())


# speedup vs baseline: 1.2895x; 1.2895x over previous
import functools

import jax
import jax.numpy as jnp
from jax import lax
from jax.experimental import pallas as pl
from jax.experimental.pallas import tpu as pltpu
from jax.experimental.pallas import tpu_sc as plsc

_N = 16384
_C = 1000

_SC_TILES_PER_TEC = 1
_NW = 32
_SC_ROWS = _SC_TILES_PER_TEC * 128 * _NW
_TC_ROWS = _N - _SC_ROWS

_BN = 2048
_NB = _TC_ROWS // _BN
_OFF = _SC_ROWS // _BN
_TR = _BN // 128
_TOFF = _SC_ROWS // 128

_CC = 200
_NCC = _C // _CC


def _tc_kernel(x_ref, t_ref, out_ref, acc_ref):
    i = pl.program_id(0)
    x = x_ref[...]
    parts = []
    for j in range(_TR):
        xc = x[:, j * 128:(j + 1) * 128]
        row = jax.lax.broadcasted_iota(jnp.int32, xc.shape, 0)
        mx = jnp.max(xc, axis=0, keepdims=True)
        parts.append(jnp.min(jnp.where(xc == mx, row, _C), axis=0, keepdims=True))
    pred = jnp.concatenate(parts, axis=0)
    tgt = t_ref[...]
    m = (pred != tgt).astype(jnp.float32)
    pf = pred.astype(jnp.float32)
    pm = pf * m
    bn = jnp.sum(m)
    bs1 = jnp.sum(pm)
    bs2 = jnp.sum(pf * pm)

    @pl.when(i == 0)
    def _():
        acc_ref[0] = bn
        acc_ref[1] = bs1
        acc_ref[2] = bs2

    @pl.when(i != 0)
    def _():
        acc_ref[0] += bn
        acc_ref[1] += bs1
        acc_ref[2] += bs2

    @pl.when(i == _NB - 1)
    def _():
        out_ref[0, 0] = acc_ref[0]
        out_ref[0, 1] = acc_ref[1]
        out_ref[0, 2] = acc_ref[2]


def _tc_partials(xt, t128):
    return pl.pallas_call(
        _tc_kernel,
        grid=(_NB,),
        in_specs=[
            pl.BlockSpec((_C, _BN), lambda i: (0, i + _OFF)),
            pl.BlockSpec((_TR, 128), lambda i: (i + _TOFF // _TR, 0)),
        ],
        out_specs=pl.BlockSpec(
            (1, 3), lambda i: (0, 0), memory_space=pltpu.SMEM
        ),
        out_shape=jax.ShapeDtypeStruct((1, 3), jnp.float32),
        scratch_shapes=[pltpu.SMEM((3,), jnp.float32)],
        compiler_params=pltpu.CompilerParams(
            dimension_semantics=("arbitrary",),
        ),
    )(xt, t128)


def _sc_partials(xt, targets):
    mesh = plsc.VectorSubcoreMesh(core_axis_name="c", subcore_axis_name="s")

    @functools.partial(
        pl.kernel,
        mesh=mesh,
        out_type=jax.ShapeDtypeStruct((_NW * 48,), jnp.float32),
        scratch_types=[
            pltpu.VMEM((_CC, 128), jnp.float32),
            pltpu.VMEM((_CC, 128), jnp.float32),
            pltpu.VMEM((128,), jnp.int32),
            pltpu.VMEM((48,), jnp.float32),
            pltpu.SemaphoreType.DMA,
            pltpu.SemaphoreType.DMA,
            pltpu.SemaphoreType.DMA,
        ],
    )
    def k(xt_hbm, t_hbm, out_hbm, buf_a, buf_b, tvec, part, sem_a, sem_b, sem_t):
        cid = lax.axis_index("c")
        sid = lax.axis_index("s")
        wid = sid * 2 + cid
        bufs = (buf_a, buf_b)
        sems = (sem_a, sem_b)

        n_acc = jnp.zeros((16,), jnp.float32)
        s1_acc = jnp.zeros((16,), jnp.float32)
        s2_acc = jnp.zeros((16,), jnp.float32)

        for k_t in range(_SC_TILES_PER_TEC):
            tile = wid * _SC_TILES_PER_TEC + k_t
            base = tile * 128
            t_h = pltpu.async_copy(
                t_hbm.at[pl.ds(base, 128)], tvec, sem_t
            )
            handles = [None, None]
            handles[0] = pltpu.async_copy(
                xt_hbm.at[pl.ds(0, _CC), pl.ds(base, 128)], buf_a, sem_a
            )
            bests = [
                (jnp.full((16,), -jnp.inf, jnp.float32),
                 jnp.zeros((16,), jnp.float32))
                for _ in range(8)
            ]
            for cc in range(_NCC):
                cur = cc % 2
                if cc + 1 < _NCC:
                    nxt = (cc + 1) % 2
                    handles[nxt] = pltpu.async_copy(
                        xt_hbm.at[pl.ds((cc + 1) * _CC, _CC), pl.ds(base, 128)],
                        bufs[nxt], sems[nxt],
                    )
                handles[cur].wait()
                buf = bufs[cur]
                for g in range(8):
                    best, bidx = bests[g]

                    def body(j, carry, _g=g, _buf=buf, _cc=cc):
                        b, bi = carry
                        for u in range(8):
                            c_loc = j * 8 + u
                            v = _buf[c_loc, pl.ds(_g * 16, 16)]
                            cf = (_cc * _CC + c_loc).astype(jnp.float32)
                            cv = jnp.full((16,), 0.0, jnp.float32) + cf
                            upd = v > b
                            b = jnp.where(upd, v, b)
                            bi = jnp.where(upd, cv, bi)
                        return b, bi

                    best, bidx = lax.fori_loop(
                        0, _CC // 8, body, (best, bidx)
                    )
                    bests[g] = (best, bidx)
            t_h.wait()
            for g in range(8):
                best, bidx = bests[g]
                tg = tvec[pl.ds(g * 16, 16)].astype(jnp.float32)
                m = jnp.where(bidx != tg, 1.0, 0.0)
                pm = bidx * m
                n_acc = n_acc + m
                s1_acc = s1_acc + pm
                s2_acc = s2_acc + bidx * pm

        part[pl.ds(0, 16)] = n_acc
        part[pl.ds(16, 16)] = s1_acc
        part[pl.ds(32, 16)] = s2_acc
        pltpu.sync_copy(part, out_hbm.at[pl.ds(wid * 48, 48)])

    return k(xt, targets)


def _combine_kernel(tc_ref, sc_ref, out_ref):
    sc = sc_ref[...]
    n = tc_ref[0, 0]
    s1 = tc_ref[0, 1]
    s2 = tc_ref[0, 2]
    nv = jnp.zeros((16,), jnp.float32)
    s1v = jnp.zeros((16,), jnp.float32)
    s2v = jnp.zeros((16,), jnp.float32)
    for w in range(_NW):
        nv = nv + lax.slice(sc, (w * 48,), (w * 48 + 16,))
        s1v = s1v + lax.slice(sc, (w * 48 + 16,), (w * 48 + 32,))
        s2v = s2v + lax.slice(sc, (w * 48 + 32,), (w * 48 + 48,))
    n = n + jnp.sum(nv)
    s1 = s1 + jnp.sum(s1v)
    s2 = s2 + jnp.sum(s2v)
    mean = s1 / n
    var = (s2 - s1 * mean) / (n - 1.0)
    out_ref[0, 0] = 1.0 - jnp.sqrt(var)


def _combine(tc_part, sc_part):
    return pl.pallas_call(
        _combine_kernel,
        in_specs=[
            pl.BlockSpec(memory_space=pltpu.SMEM),
            pl.BlockSpec(memory_space=pltpu.VMEM),
        ],
        out_specs=pl.BlockSpec(memory_space=pltpu.SMEM),
        out_shape=jax.ShapeDtypeStruct((1, 1), jnp.float32),
    )(tc_part, sc_part)


def kernel(inputs, targets):
    xt = inputs.T
    t128 = targets.reshape(128, 128)
    sc_part = _sc_partials(xt, targets)
    tc_part = _tc_partials(xt, t128)
    out = _combine(tc_part, sc_part)
    return out.reshape(())


# speedup vs baseline: 2.0958x; 1.6253x over previous
import jax
import jax.numpy as jnp
from jax.experimental import pallas as pl
from jax.experimental.pallas import tpu as pltpu

_N = 16384
_C = 1000
_BN = 2048
_NB = _N // _BN
_TR = _BN // 128


def _dl_kernel(x_ref, t_ref, out_ref, acc_ref):
    i = pl.program_id(0)
    x = x_ref[...]
    parts = []
    for j in range(_TR):
        xc = x[:, j * 128:(j + 1) * 128]
        row = jax.lax.broadcasted_iota(jnp.int32, xc.shape, 0)
        mx = jnp.max(xc, axis=0, keepdims=True)
        parts.append(jnp.min(jnp.where(xc == mx, row, _C), axis=0, keepdims=True))
    pred = jnp.concatenate(parts, axis=0)
    tgt = t_ref[...]
    m = (pred != tgt).astype(jnp.float32)
    pf = pred.astype(jnp.float32)
    pm = pf * m
    bn = jnp.sum(m)
    bs1 = jnp.sum(pm)
    bs2 = jnp.sum(pf * pm)

    @pl.when(i == 0)
    def _():
        acc_ref[0] = bn
        acc_ref[1] = bs1
        acc_ref[2] = bs2

    @pl.when(i != 0)
    def _():
        acc_ref[0] += bn
        acc_ref[1] += bs1
        acc_ref[2] += bs2

    @pl.when(i == _NB - 1)
    def _():
        n = acc_ref[0]
        s1 = acc_ref[1]
        s2 = acc_ref[2]
        mean = s1 / n
        var = (s2 - s1 * mean) / (n - 1.0)
        out_ref[0, 0] = 1.0 - jnp.sqrt(var)


def kernel(inputs, targets):
    xt = inputs.T
    t128 = targets.reshape(128, 128)
    out = pl.pallas_call(
        _dl_kernel,
        grid=(_NB,),
        in_specs=[
            pl.BlockSpec((_C, _BN), lambda i: (0, i)),
            pl.BlockSpec((_TR, 128), lambda i: (i, 0)),
        ],
        out_specs=pl.BlockSpec(
            (1, 1), lambda i: (0, 0), memory_space=pltpu.SMEM
        ),
        out_shape=jax.ShapeDtypeStruct((1, 1), jnp.float32),
        scratch_shapes=[pltpu.SMEM((3,), jnp.float32)],
        compiler_params=pltpu.CompilerParams(
            dimension_semantics=("arbitrary",),
        ),
    )(xt, t128)
    return out.reshape(())


# speedup vs baseline: 2.1064x; 1.0051x over previous
import jax
import jax.numpy as jnp
from jax.experimental import pallas as pl
from jax.experimental.pallas import tpu as pltpu

_N = 16384
_C = 1000
_BN = 2048
_H = _BN // 2
_NB = _N // _BN
_TR = _BN // 128


def _dl_kernel(a_ref, b_ref, t_ref, out_ref, acc_ref):
    i = pl.program_id(0)
    parts = []
    for h, ref in ((0, a_ref), (1, b_ref)):
        x = ref[...]
        for j in range(_H // 128):
            xc = x[:, j * 128:(j + 1) * 128]
            row = jax.lax.broadcasted_iota(jnp.int32, xc.shape, 0)
            mx = jnp.max(xc, axis=0, keepdims=True)
            parts.append(
                jnp.min(jnp.where(xc == mx, row, _C), axis=0, keepdims=True)
            )
    pred = jnp.concatenate(parts, axis=0)
    tgt = t_ref[...]
    m = (pred != tgt).astype(jnp.float32)
    pf = pred.astype(jnp.float32)
    pm = pf * m
    bn = jnp.sum(m)
    bs1 = jnp.sum(pm)
    bs2 = jnp.sum(pf * pm)

    @pl.when(i == 0)
    def _():
        acc_ref[0] = bn
        acc_ref[1] = bs1
        acc_ref[2] = bs2

    @pl.when(i != 0)
    def _():
        acc_ref[0] += bn
        acc_ref[1] += bs1
        acc_ref[2] += bs2

    @pl.when(i == _NB - 1)
    def _():
        n = acc_ref[0]
        s1 = acc_ref[1]
        s2 = acc_ref[2]
        mean = s1 / n
        var = (s2 - s1 * mean) / (n - 1.0)
        out_ref[0, 0] = 1.0 - jnp.sqrt(var)


def kernel(inputs, targets):
    xt = inputs.T
    t128 = targets.reshape(128, 128)
    out = pl.pallas_call(
        _dl_kernel,
        grid=(_NB,),
        in_specs=[
            pl.BlockSpec((_C, _H), lambda i: (0, 2 * i)),
            pl.BlockSpec((_C, _H), lambda i: (0, 2 * i + 1)),
            pl.BlockSpec((_TR, 128), lambda i: (i, 0)),
        ],
        out_specs=pl.BlockSpec(
            (1, 1), lambda i: (0, 0), memory_space=pltpu.SMEM
        ),
        out_shape=jax.ShapeDtypeStruct((1, 1), jnp.float32),
        scratch_shapes=[pltpu.SMEM((3,), jnp.float32)],
        compiler_params=pltpu.CompilerParams(
            dimension_semantics=("arbitrary",),
        ),
    )(xt, xt, t128)
    return out.reshape(())
